# baseline (speedup 1.0000x reference)
import functools

import jax
import jax.numpy as jnp
from jax import lax
from jax.experimental import pallas as pl
from jax.experimental.pallas import tpu as pltpu
from jax.experimental.pallas import tpu_sc as plsc

B = 16384
D = 64
K = 5
NODE = 1000000
NP = NODE // 2
NC = 2
NS = 16
NW = NC * NS
PW = B // NW
CH = 128
NCH = PW // CH
L = 16
G = CH // L

TCW = 16384
TGRID = -(-NODE // (2 * TCW))
NPAD = TGRID * TCW
HSH = TCW.bit_length() - 1


def _pack_body(x_ref, o_ref):
    x = x_ref[...]
    xcat = jnp.concatenate([x[:, :TCW], x[:, TCW:]], axis=0)
    ident = (lax.broadcasted_iota(jnp.int32, (2 * D, 2 * D), 0)
             == lax.broadcasted_iota(jnp.int32, (2 * D, 2 * D), 1)
             ).astype(jnp.float32)
    dn = (((0,), (0,)), ((), ()))
    o_ref[...] = lax.dot_general(
        xcat, ident, dn, preferred_element_type=jnp.float32
    ).astype(jnp.bfloat16)


def _pack_pairs(table_t):
    return pl.pallas_call(
        _pack_body,
        grid=(TGRID,),
        in_specs=[pl.BlockSpec((D, 2 * TCW), lambda j: (0, j))],
        out_specs=pl.BlockSpec((TCW, 2 * D), lambda j: (j, 0)),
        out_shape=jax.ShapeDtypeStruct((NPAD, 2 * D), jnp.bfloat16),
    )(table_t)


def _sc_scores(idx_all, u_pack, v_pack):
    mesh = plsc.VectorSubcoreMesh(core_axis_name="c", subcore_axis_name="s")

    @functools.partial(
        pl.kernel,
        out_type=jax.ShapeDtypeStruct((NW, 1 + K, PW), jnp.float32),
        mesh=mesh,
        scratch_types=[
            pltpu.VMEM(((2 + K) * PW,), jnp.int32),
            pltpu.VMEM(((2 + K) * PW,), jnp.int32),
            pltpu.VMEM((2 * CH, D // 2), jnp.int32),
            pltpu.VMEM((2 * CH, D // 2), jnp.int32),
            pltpu.VMEM((2 * K * CH, D // 2), jnp.int32),
            pltpu.VMEM((1 + K, PW), jnp.float32),
            pltpu.SemaphoreType.DMA,
            pltpu.SemaphoreType.DMA,
            pltpu.SemaphoreType.DMA,
            pltpu.SemaphoreType.DMA,
            pltpu.SemaphoreType.DMA,
            pltpu.SemaphoreType.DMA,
        ],
        compiler_params=pltpu.CompilerParams(
            needs_layout_passes=False, use_tc_tiling_on_sc=False),
    )
    def body(idx_hbm, uw_hbm, vw_hbm, out_hbm, idx_v, idxp_v, ru, rv, rn, sbuf,
             *sems):
        wid = lax.axis_index("s") * NC + lax.axis_index("c")
        pltpu.sync_copy(idx_hbm.at[wid], idx_v)
        for i in range((2 + K) * PW // L):
            raw = idx_v[pl.ds(i * L, L)]
            q = ((raw >> (HSH + 1)) << HSH) | (raw & (TCW - 1))
            idxp_v[pl.ds(i * L, L)] = (q << 1) | ((raw >> HSH) & 1)
        iota = lax.iota(jnp.int32, L)

        def issue(c, b):
            cu = pltpu.async_copy(
                uw_hbm.at[idxp_v.at[pl.ds(c * CH, CH)]],
                ru.at[pl.ds(b * CH, CH)], sems[b])
            cv = pltpu.async_copy(
                vw_hbm.at[idxp_v.at[pl.ds(PW + c * CH, CH)]],
                rv.at[pl.ds(b * CH, CH)], sems[2 + b])
            cns = [
                pltpu.async_copy(
                    vw_hbm.at[idxp_v.at[pl.ds((2 + k) * PW + c * CH, CH)]],
                    rn.at[pl.ds((b * K + k) * CH, CH)], sems[4 + b])
                for k in range(K)
            ]
            return [cu, cv] + cns

        pending = {0: issue(0, 0)}
        for c in range(NCH):
            b = c & 1
            if c + 1 < NCH:
                pending[c + 1] = issue(c + 1, 1 - b)
            for cp in pending.pop(c):
                cp.wait()
            for g in range(G):
                rows = b * K * CH + g * L + iota
                rowuv = b * CH + g * L + iota
                off = c * CH + g * L
                zero = jnp.zeros((L,), jnp.float32)

                def unpk(w):
                    return plsc.unpack(plsc.bitcast(w, jnp.bfloat16),
                                       format=plsc.PackFormat.INTERLEAVED)

                def dbody(dd, accs, rows=rows, rowuv=rowuv):
                    base = jnp.zeros((L,), jnp.int32) + dd * 4
                    for t in range(4):
                        dvec = base + t
                        ua, ub = unpk(plsc.load_gather(ru, [rowuv, dvec]))
                        va, vb = unpk(plsc.load_gather(rv, [rowuv, dvec]))
                        s0 = accs[0] + ua * va + ub * vb
                        ss = []
                        for k in range(K):
                            na, nb = unpk(
                                plsc.load_gather(rn, [k * CH + rows, dvec]))
                            ss.append(accs[1 + k] + ua * na + ub * nb)
                        accs = (s0, *ss)
                    return accs

                accs = lax.fori_loop(0, D // 8, dbody, (zero,) * (1 + K))
                sbuf[0, pl.ds(off, L)] = -accs[0]
                for k in range(K):
                    sbuf[1 + k, pl.ds(off, L)] = accs[1 + k]
        pltpu.sync_copy(sbuf, out_hbm.at[wid])

    return body(idx_all, u_pack, v_pack)


def _loss_body(x_ref, o_ref):
    x = jnp.clip(x_ref[...], -10.0, 10.0)
    o_ref[...] = (jnp.sum(jnp.log1p(jnp.exp(x))) * (1.0 / B)).reshape(1, 1)


def kernel(pos_u, pos_v, neg_v, u_weight, v_weight):
    idx_all = jnp.concatenate(
        [pos_u[None, :], pos_v[None, :], neg_v.T.astype(jnp.int32)], axis=0)
    idx_all = idx_all.reshape(2 + K, NW, PW).transpose(1, 0, 2).reshape(
        NW, (2 + K) * PW)
    def _as_i32_rows(p):
        pi = lax.bitcast_convert_type(p.reshape(NPAD, D, 2), jnp.int32)
        return pi.reshape(2 * NPAD, D // 2)

    u_pack = _as_i32_rows(_pack_pairs(u_weight.T))
    v_pack = _as_i32_rows(_pack_pairs(v_weight.T))
    scores = _sc_scores(idx_all, u_pack, v_pack)
    loss = pl.pallas_call(
        _loss_body,
        out_shape=jax.ShapeDtypeStruct((1, 1), jnp.float32),
    )(scores.reshape(NW * (1 + K) * PW // 2048, 2048))
    return loss[0, 0]


# speedup vs baseline: 6.3679x; 6.3679x over previous
import functools

import jax
import jax.numpy as jnp
from jax import lax
from jax.experimental import pallas as pl
from jax.experimental.pallas import tpu as pltpu
from jax.experimental.pallas import tpu_sc as plsc

B = 16384
D = 64
K = 5
NODE = 1000000
NP = NODE // 2
NC = 2
NS = 16
NW = NC * NS
PW = B // NW
CH = 128
NCH = PW // CH
L = 16
G = CH // L

TCW = 8192
TGRID = -(-NODE // (4 * TCW))
NQ = TGRID * TCW
HSH = TCW.bit_length() - 1


def _bf16_hi(y):
    u = lax.bitcast_convert_type(y, jnp.int32)
    return u + (((u >> 16) & 1) + 0x7FFF)


def _pack_body(x_ref, o_ref):
    x = x_ref[...]
    ident = (lax.broadcasted_iota(jnp.int32, (2 * D, 2 * D), 0)
             == lax.broadcasted_iota(jnp.int32, (2 * D, 2 * D), 1)
             ).astype(jnp.float32)
    dn = (((0,), (0,)), ((), ()))
    zs = []
    for half in range(2):
        xc = jnp.concatenate(
            [x[:, (2 * half) * TCW:(2 * half + 1) * TCW],
             x[:, (2 * half + 1) * TCW:(2 * half + 2) * TCW]], axis=0)
        y = lax.dot_general(xc, ident, dn,
                            preferred_element_type=jnp.float32)
        for p in range(2):
            lo = (_bf16_hi(y[:, p * D:p * D + 32]) >> 16) & 0xFFFF
            hi = _bf16_hi(y[:, p * D + 32:(p + 1) * D]) & jnp.int32(-65536)
            zs.append(hi | lo)
    o_ref[...] = jnp.concatenate(zs, axis=1)


def _pack_rows(table_t):
    return pl.pallas_call(
        _pack_body,
        grid=(TGRID,),
        in_specs=[pl.BlockSpec((D, 4 * TCW), lambda j: (0, j))],
        out_specs=pl.BlockSpec((TCW, 2 * D), lambda j: (j, 0)),
        out_shape=jax.ShapeDtypeStruct((NQ, 2 * D), jnp.int32),
    )(table_t)


def _sc_scores(idx_all, u_pack, v_pack):
    mesh = plsc.VectorSubcoreMesh(core_axis_name="c", subcore_axis_name="s")

    @functools.partial(
        pl.kernel,
        out_type=jax.ShapeDtypeStruct((NW, 1 + K, PW), jnp.float32),
        mesh=mesh,
        scratch_types=[
            pltpu.VMEM(((2 + K) * PW,), jnp.int32),
            pltpu.VMEM(((2 + K) * PW,), jnp.int32),
            pltpu.VMEM((2 * CH, D // 2), jnp.int32),
            pltpu.VMEM((2 * CH, D // 2), jnp.int32),
            pltpu.VMEM((2 * K * CH, D // 2), jnp.int32),
            pltpu.VMEM((1 + K, PW), jnp.float32),
            pltpu.SemaphoreType.DMA,
            pltpu.SemaphoreType.DMA,
            pltpu.SemaphoreType.DMA,
            pltpu.SemaphoreType.DMA,
            pltpu.SemaphoreType.DMA,
            pltpu.SemaphoreType.DMA,
        ],
        compiler_params=pltpu.CompilerParams(
            needs_layout_passes=False, use_tc_tiling_on_sc=False),
    )
    def body(idx_hbm, uw_hbm, vw_hbm, out_hbm, idx_v, idxp_v, ru, rv, rn, sbuf,
             *sems):
        wid = lax.axis_index("s") * NC + lax.axis_index("c")
        pltpu.sync_copy(idx_hbm.at[wid], idx_v)
        for i in range((2 + K) * PW // L):
            raw = idx_v[pl.ds(i * L, L)]
            q = ((raw >> (HSH + 2)) << HSH) | (raw & (TCW - 1))
            idxp_v[pl.ds(i * L, L)] = (q << 2) | ((raw >> HSH) & 3)
        iota = lax.iota(jnp.int32, L)

        def issue(c, b):
            cu = pltpu.async_copy(
                uw_hbm.at[idxp_v.at[pl.ds(c * CH, CH)]],
                ru.at[pl.ds(b * CH, CH)], sems[b])
            cv = pltpu.async_copy(
                vw_hbm.at[idxp_v.at[pl.ds(PW + c * CH, CH)]],
                rv.at[pl.ds(b * CH, CH)], sems[2 + b])
            cns = [
                pltpu.async_copy(
                    vw_hbm.at[idxp_v.at[pl.ds((2 + k) * PW + c * CH, CH)]],
                    rn.at[pl.ds((b * K + k) * CH, CH)], sems[4 + b])
                for k in range(K)
            ]
            return [cu, cv] + cns

        pending = {0: issue(0, 0)}
        for c in range(NCH):
            b = c & 1
            if c + 1 < NCH:
                pending[c + 1] = issue(c + 1, 1 - b)
            for cp in pending.pop(c):
                cp.wait()
            for g in range(G):
                rows = b * K * CH + g * L + iota
                rowuv = b * CH + g * L + iota
                off = c * CH + g * L
                zero = jnp.zeros((L,), jnp.float32)

                def unpk(w):
                    a = plsc.bitcast(w << 16, jnp.float32)
                    b = plsc.bitcast(w & jnp.int32(-65536), jnp.float32)
                    return a, b

                def dbody(dd, accs, rows=rows, rowuv=rowuv):
                    base = jnp.zeros((L,), jnp.int32) + dd * 4
                    for t in range(4):
                        dvec = base + t
                        ua, ub = unpk(plsc.load_gather(ru, [rowuv, dvec]))
                        va, vb = unpk(plsc.load_gather(rv, [rowuv, dvec]))
                        s0 = accs[0] + ua * va + ub * vb
                        ss = []
                        for k in range(K):
                            na, nb = unpk(
                                plsc.load_gather(rn, [k * CH + rows, dvec]))
                            ss.append(accs[1 + k] + ua * na + ub * nb)
                        accs = (s0, *ss)
                    return accs

                accs = lax.fori_loop(0, D // 8, dbody, (zero,) * (1 + K))
                sbuf[0, pl.ds(off, L)] = -accs[0]
                for k in range(K):
                    sbuf[1 + k, pl.ds(off, L)] = accs[1 + k]
        pltpu.sync_copy(sbuf, out_hbm.at[wid])

    return body(idx_all, u_pack, v_pack)


def _loss_body(x_ref, o_ref):
    x = jnp.clip(x_ref[...], -10.0, 10.0)
    o_ref[...] = (jnp.sum(jnp.log1p(jnp.exp(x))) * (1.0 / B)).reshape(1, 1)


def kernel(pos_u, pos_v, neg_v, u_weight, v_weight):
    idx_all = jnp.concatenate(
        [pos_u[None, :], pos_v[None, :], neg_v.T.astype(jnp.int32)], axis=0)
    idx_all = idx_all.reshape(2 + K, NW, PW).transpose(1, 0, 2).reshape(
        NW, (2 + K) * PW)
    u_pack = _pack_rows(u_weight.T).reshape(4 * NQ, D // 2)
    v_pack = _pack_rows(v_weight.T).reshape(4 * NQ, D // 2)
    scores = _sc_scores(idx_all, u_pack, v_pack)
    loss = pl.pallas_call(
        _loss_body,
        out_shape=jax.ShapeDtypeStruct((1, 1), jnp.float32),
    )(scores.reshape(NW * (1 + K) * PW // 2048, 2048))
    return loss[0, 0]


# speedup vs baseline: 7.3320x; 1.1514x over previous
import functools

import jax
import jax.numpy as jnp
from jax import lax
from jax.experimental import pallas as pl
from jax.experimental.pallas import tpu as pltpu
from jax.experimental.pallas import tpu_sc as plsc

B = 16384
D = 64
K = 5
NODE = 1000000
NC = 2
NS = 16
NW = NC * NS
PW = B // NW
CH = 128
NCH = PW // CH
L = 16
G = CH // L

TCW = 16384
TGRID = -(-NODE // (2 * TCW))
NPAD = TGRID * TCW
HSH = TCW.bit_length() - 1


def _pack_body(x_ref, o_ref):
    x = x_ref[...]
    xcat = jnp.concatenate([x[:, :TCW], x[:, TCW:]], axis=0)
    ident = (lax.broadcasted_iota(jnp.int32, (2 * D, 2 * D), 0)
             == lax.broadcasted_iota(jnp.int32, (2 * D, 2 * D), 1)
             ).astype(jnp.float32)
    dn = (((0,), (0,)), ((), ()))
    o_ref[...] = lax.dot_general(xcat, ident, dn,
                                 preferred_element_type=jnp.float32)


def _pack_pairs(table_t):
    return pl.pallas_call(
        _pack_body,
        grid=(TGRID,),
        in_specs=[pl.BlockSpec((D, 2 * TCW), lambda j: (0, j))],
        out_specs=pl.BlockSpec((TCW, 2 * D), lambda j: (j, 0)),
        out_shape=jax.ShapeDtypeStruct((NPAD, 2 * D), jnp.float32),
    )(table_t)


def _sc_scores(idx_all, u_pack, v_pack):
    mesh = plsc.VectorSubcoreMesh(core_axis_name="c", subcore_axis_name="s")

    @functools.partial(
        pl.kernel,
        out_type=jax.ShapeDtypeStruct((NW, 1 + K, PW), jnp.float32),
        mesh=mesh,
        scratch_types=[
            pltpu.VMEM(((2 + K) * PW,), jnp.int32),
            pltpu.VMEM(((2 + K) * PW,), jnp.int32),
            pltpu.VMEM((2 * CH, D), jnp.float32),
            pltpu.VMEM((2 * CH, D), jnp.float32),
            pltpu.VMEM((2 * K * CH, D), jnp.float32),
            pltpu.VMEM((1 + K, PW), jnp.float32),
            pltpu.SemaphoreType.DMA,
            pltpu.SemaphoreType.DMA,
            pltpu.SemaphoreType.DMA,
            pltpu.SemaphoreType.DMA,
            pltpu.SemaphoreType.DMA,
            pltpu.SemaphoreType.DMA,
        ],
        compiler_params=pltpu.CompilerParams(
            needs_layout_passes=False, use_tc_tiling_on_sc=False),
    )
    def body(idx_hbm, uw_hbm, vw_hbm, out_hbm, idx_v, idxp_v, ru, rv, rn, sbuf,
             *sems):
        wid = lax.axis_index("s") * NC + lax.axis_index("c")
        pltpu.sync_copy(idx_hbm.at[wid], idx_v)
        for i in range((2 + K) * PW // L):
            raw = idx_v[pl.ds(i * L, L)]
            q = ((raw >> (HSH + 1)) << HSH) | (raw & (TCW - 1))
            idxp_v[pl.ds(i * L, L)] = (q << 1) | ((raw >> HSH) & 1)
        iota = lax.iota(jnp.int32, L)

        def issue(c, b):
            cu = pltpu.async_copy(
                uw_hbm.at[idxp_v.at[pl.ds(c * CH, CH)]],
                ru.at[pl.ds(b * CH, CH)], sems[b])
            cv = pltpu.async_copy(
                vw_hbm.at[idxp_v.at[pl.ds(PW + c * CH, CH)]],
                rv.at[pl.ds(b * CH, CH)], sems[2 + b])
            cns = [
                pltpu.async_copy(
                    vw_hbm.at[idxp_v.at[pl.ds((2 + k) * PW + c * CH, CH)]],
                    rn.at[pl.ds((b * K + k) * CH, CH)], sems[4 + b])
                for k in range(K)
            ]
            return [cu, cv] + cns

        pending = {0: issue(0, 0)}
        for c in range(NCH):
            b = c & 1
            if c + 1 < NCH:
                pending[c + 1] = issue(c + 1, 1 - b)
            for cp in pending.pop(c):
                cp.wait()
            for g in range(G):
                rows = b * K * CH + g * L + iota
                rowuv = b * CH + g * L + iota
                off = c * CH + g * L
                zero = jnp.zeros((L,), jnp.float32)

                def dbody(dd, accs, rows=rows, rowuv=rowuv):
                    base = jnp.zeros((L,), jnp.int32) + dd * 4
                    for t in range(4):
                        dvec = base + t
                        u = plsc.load_gather(ru, [rowuv, dvec])
                        v = plsc.load_gather(rv, [rowuv, dvec])
                        s0 = accs[0] + u * v
                        ss = [
                            accs[1 + k]
                            + u * plsc.load_gather(rn, [k * CH + rows, dvec])
                            for k in range(K)
                        ]
                        accs = (s0, *ss)
                    return accs

                accs = lax.fori_loop(0, D // 4, dbody, (zero,) * (1 + K))
                sbuf[0, pl.ds(off, L)] = -accs[0]
                for k in range(K):
                    sbuf[1 + k, pl.ds(off, L)] = accs[1 + k]
        pltpu.sync_copy(sbuf, out_hbm.at[wid])

    return body(idx_all, u_pack, v_pack)


def _loss_body(x_ref, o_ref):
    x = jnp.clip(x_ref[...], -10.0, 10.0)
    o_ref[...] = (jnp.sum(jnp.log1p(jnp.exp(x))) * (1.0 / B)).reshape(1, 1)


def kernel(pos_u, pos_v, neg_v, u_weight, v_weight):
    idx_all = jnp.concatenate(
        [pos_u[None, :], pos_v[None, :], neg_v.T.astype(jnp.int32)], axis=0)
    idx_all = idx_all.reshape(2 + K, NW, PW).transpose(1, 0, 2).reshape(
        NW, (2 + K) * PW)
    u_pack = _pack_pairs(u_weight.T).reshape(2 * NPAD, D)
    v_pack = _pack_pairs(v_weight.T).reshape(2 * NPAD, D)
    scores = _sc_scores(idx_all, u_pack, v_pack)
    loss = pl.pallas_call(
        _loss_body,
        out_shape=jax.ShapeDtypeStruct((1, 1), jnp.float32),
    )(scores.reshape(NW * (1 + K) * PW // 2048, 2048))
    return loss[0, 0]
